# baseline (speedup 1.0000x reference)
import functools

import jax
import jax.numpy as jnp
from jax import lax
from jax.experimental import pallas as pl
from jax.experimental.pallas import tpu as pltpu
from jax.experimental.pallas import tpu_sc as plsc

B = 16384
L = 200
VOCAB = 7800
DIM = 64
VPAD = 7808
NC, NS = 2, 16
NW = NC * NS
RPW = B // NW
GROUPS = RPW // 16
CHUNK = 256


def _project_body(e_ref, w_ref, o_ref):
    o_ref[...] = jnp.sum(e_ref[...] * w_ref[...], axis=1, keepdims=True)


def _project(table_pad, lin_w):
    return pl.pallas_call(
        _project_body,
        out_shape=jax.ShapeDtypeStruct((VPAD, 1), jnp.float32),
    )(table_pad, lin_w)


_MESH = plsc.VectorSubcoreMesh(core_axis_name="c", subcore_axis_name="s")


@functools.partial(
    pl.kernel,
    out_type=jax.ShapeDtypeStruct((B,), jnp.float32),
    mesh=_MESH,
    compiler_params=pltpu.CompilerParams(needs_layout_passes=False),
    scratch_types=[
        pltpu.VMEM((VPAD,), jnp.float32),
        pltpu.VMEM((CHUNK, L), jnp.int32),
        pltpu.VMEM((RPW,), jnp.float32),
        pltpu.VMEM((16,), jnp.float32),
        pltpu.VMEM((RPW,), jnp.float32),
    ],
)
def _sc_pool(s_hbm, x_hbm, len_hbm, bias_hbm, out_hbm,
             s_v, x_v, len_v, bias_v, out_v):
    wid = lax.axis_index("s") * NC + lax.axis_index("c")
    base = wid * RPW
    pltpu.sync_copy(s_hbm, s_v)
    pltpu.sync_copy(len_hbm.at[pl.ds(base, RPW)], len_v)
    pltpu.sync_copy(bias_hbm, bias_v)

    lanes = lax.iota(jnp.int32, 16)
    bias = bias_v[...]
    UNROLL = 8
    zero = jnp.zeros((16,), jnp.float32)

    for half in range(RPW // CHUNK):
        pltpu.sync_copy(x_hbm.at[pl.ds(base + half * CHUNK, CHUNK), :], x_v)

        def group_body(g, carry):
            rows = g * 16 + lanes

            def inner(i, st):
                acc0, acc1 = st
                c0 = i * UNROLL
                for k in range(UNROLL):
                    cols = jnp.full((16,), c0 + k, jnp.int32)
                    xv = plsc.load_gather(x_v, [rows, cols])
                    val = plsc.load_gather(s_v, [xv])
                    if k % 2 == 0:
                        acc0 = acc0 + val
                    else:
                        acc1 = acc1 + val
                return acc0, acc1

            a0, a1 = lax.fori_loop(0, L // UNROLL, inner, (zero, zero))
            off = half * CHUNK + g * 16
            z = (a0 + a1) / len_v[pl.ds(off, 16)] + bias
            out_v[pl.ds(off, 16)] = 1.0 / (1.0 + jnp.exp(-z))
            return carry

        lax.fori_loop(0, CHUNK // 16, group_body, 0)
    pltpu.sync_copy(out_v, out_hbm.at[pl.ds(base, RPW)])


def kernel(x, length, embed_table, lin_w, lin_b):
    table_pad = jnp.pad(embed_table, ((0, VPAD - VOCAB), (0, 0)))
    s = _project(table_pad, lin_w).reshape(VPAD)
    bias16 = jnp.broadcast_to(lin_b, (16,)).astype(jnp.float32)
    y = _sc_pool(s, x, length, bias16)
    return y.reshape(B, 1)


# speedup vs baseline: 1.0565x; 1.0565x over previous
import functools

import jax
import jax.numpy as jnp
from jax import lax
from jax.experimental import pallas as pl
from jax.experimental.pallas import tpu as pltpu
from jax.experimental.pallas import tpu_sc as plsc

B = 16384
L = 200
VOCAB = 7800
DIM = 64
VPAD = 7808
NC, NS = 2, 16
NW = NC * NS
RPW = B // NW
GROUPS = RPW // 16
DCH = 64


def _project_body(e_ref, w_ref, o_ref):
    o_ref[...] = jnp.sum(e_ref[...] * w_ref[...], axis=1, keepdims=True)


def _project(table_pad, lin_w):
    return pl.pallas_call(
        _project_body,
        out_shape=jax.ShapeDtypeStruct((VPAD, 1), jnp.float32),
    )(table_pad, lin_w)


_MESH = plsc.VectorSubcoreMesh(core_axis_name="c", subcore_axis_name="s")


@functools.partial(
    pl.kernel,
    out_type=jax.ShapeDtypeStruct((B,), jnp.float32),
    mesh=_MESH,
    compiler_params=pltpu.CompilerParams(needs_layout_passes=False),
    scratch_types=[
        pltpu.VMEM((VPAD,), jnp.float32),
        pltpu.VMEM((DCH, L), jnp.int32),
        pltpu.VMEM((DCH, L), jnp.int32),
        pltpu.VMEM((RPW,), jnp.float32),
        pltpu.VMEM((16,), jnp.float32),
        pltpu.VMEM((RPW,), jnp.float32),
        pltpu.SemaphoreType.DMA,
        pltpu.SemaphoreType.DMA,
    ],
)
def _sc_pool(s_hbm, x_hbm, len_hbm, bias_hbm, out_hbm,
             s_v, xb0, xb1, len_v, bias_v, out_v, sem0, sem1):
    wid = lax.axis_index("s") * NC + lax.axis_index("c")
    base = wid * RPW
    pltpu.sync_copy(s_hbm, s_v)
    pltpu.sync_copy(len_hbm.at[pl.ds(base, RPW)], len_v)
    pltpu.sync_copy(bias_hbm, bias_v)

    lanes = lax.iota(jnp.int32, 16)
    bias = bias_v[...]
    UNROLL = 8
    zero = jnp.zeros((16,), jnp.float32)
    zero_i = jnp.zeros((16,), jnp.int32)
    bufs = (xb0, xb1)
    sems = (sem0, sem1)
    nchunk = RPW // DCH

    def start(c):
        return pltpu.async_copy(
            x_hbm.at[pl.ds(base + c * DCH, DCH), :], bufs[c % 2], sems[c % 2])

    cp = start(0)
    for c in range(nchunk):
        cp.wait()
        if c + 1 < nchunk:
            cp = start(c + 1)
        x_v = bufs[c % 2]

        def group_body(g, carry):
            rows = g * 16 + lanes

            def inner(_, st):
                acc0, acc1, cols = st
                for k in range(UNROLL):
                    xv = plsc.load_gather(x_v, [rows, cols + k])
                    val = plsc.load_gather(s_v, [xv])
                    if k % 2 == 0:
                        acc0 = acc0 + val
                    else:
                        acc1 = acc1 + val
                return acc0, acc1, cols + UNROLL

            a0, a1, _ = lax.fori_loop(
                0, L // UNROLL, inner, (zero, zero, zero_i))
            off = c * DCH + g * 16
            z = (a0 + a1) / len_v[pl.ds(off, 16)] + bias
            out_v[pl.ds(off, 16)] = 1.0 / (1.0 + jnp.exp(-z))
            return carry

        lax.fori_loop(0, DCH // 16, group_body, 0)
    pltpu.sync_copy(out_v, out_hbm.at[pl.ds(base, RPW)])


def kernel(x, length, embed_table, lin_w, lin_b):
    table_pad = jnp.pad(embed_table, ((0, VPAD - VOCAB), (0, 0)))
    s = _project(table_pad, lin_w).reshape(VPAD)
    bias16 = jnp.broadcast_to(lin_b, (16,)).astype(jnp.float32)
    y = _sc_pool(s, x, length, bias16)
    return y.reshape(B, 1)


# speedup vs baseline: 1.3126x; 1.2423x over previous
import functools

import jax
import jax.numpy as jnp
from jax import lax
from jax.experimental import pallas as pl
from jax.experimental.pallas import tpu as pltpu
from jax.experimental.pallas import tpu_sc as plsc

B = 16384
L = 200
VOCAB = 7800
DIM = 64
VPAD = 7808
NC, NS = 2, 16
NW = NC * NS
RPW = B // NW
GROUPS = RPW // 16
DCH = 64


def _project_body(e_ref, w_ref, o_ref):
    o_ref[...] = jnp.sum(e_ref[...] * w_ref[...], axis=1, keepdims=True)


def _project(table_pad, lin_w):
    return pl.pallas_call(
        _project_body,
        out_shape=jax.ShapeDtypeStruct((VPAD, 1), jnp.float32),
    )(table_pad, lin_w)


_MESH = plsc.VectorSubcoreMesh(core_axis_name="c", subcore_axis_name="s")


@functools.partial(
    pl.kernel,
    out_type=jax.ShapeDtypeStruct((B,), jnp.float32),
    mesh=_MESH,
    compiler_params=pltpu.CompilerParams(needs_layout_passes=False),
    scratch_types=[
        pltpu.VMEM((VPAD,), jnp.float32),
        pltpu.VMEM((DCH * L,), jnp.int32),
        pltpu.VMEM((DCH * L,), jnp.int32),
        pltpu.VMEM((RPW,), jnp.float32),
        pltpu.VMEM((16,), jnp.float32),
        pltpu.VMEM((RPW,), jnp.float32),
        pltpu.SemaphoreType.DMA,
        pltpu.SemaphoreType.DMA,
    ],
)
def _sc_pool(s_hbm, x_hbm, len_hbm, bias_hbm, out_hbm,
             s_v, xb0, xb1, len_v, bias_v, out_v, sem0, sem1):
    wid = lax.axis_index("s") * NC + lax.axis_index("c")
    base = wid * RPW
    pltpu.sync_copy(s_hbm, s_v)
    pltpu.sync_copy(len_hbm.at[pl.ds(base, RPW)], len_v)
    pltpu.sync_copy(bias_hbm, bias_v)

    lanes = lax.iota(jnp.int32, 16)
    bias = bias_v[...]
    UNROLL = 8
    zero = jnp.zeros((16,), jnp.float32)
    zero_i = jnp.zeros((16,), jnp.int32)
    bufs = (xb0, xb1)
    sems = (sem0, sem1)
    nchunk = RPW // DCH

    def start(c):
        return pltpu.async_copy(
            x_hbm.at[pl.ds((base + c * DCH) * L, DCH * L)],
            bufs[c % 2], sems[c % 2])

    cp = start(0)
    for c in range(nchunk):
        cp.wait()
        if c + 1 < nchunk:
            cp = start(c + 1)
        x_v = bufs[c % 2]

        def group_body(g, carry):
            idx0 = (g * 16 + lanes) * L

            def inner(_, st):
                acc0, acc1, idx = st
                for k in range(UNROLL):
                    xv = plsc.load_gather(x_v, [idx + k])
                    val = plsc.load_gather(s_v, [xv])
                    if k % 2 == 0:
                        acc0 = acc0 + val
                    else:
                        acc1 = acc1 + val
                return acc0, acc1, idx + UNROLL

            a0, a1, _ = lax.fori_loop(
                0, L // UNROLL, inner, (zero, zero, idx0))
            off = c * DCH + g * 16
            z = (a0 + a1) / len_v[pl.ds(off, 16)] + bias
            out_v[pl.ds(off, 16)] = 1.0 / (1.0 + jnp.exp(-z))
            return carry

        lax.fori_loop(0, DCH // 16, group_body, 0)
    pltpu.sync_copy(out_v, out_hbm.at[pl.ds(base, RPW)])


def kernel(x, length, embed_table, lin_w, lin_b):
    table_pad = jnp.pad(embed_table, ((0, VPAD - VOCAB), (0, 0)))
    s = _project(table_pad, lin_w).reshape(VPAD)
    bias16 = jnp.broadcast_to(lin_b, (16,)).astype(jnp.float32)
    y = _sc_pool(s, x.reshape(-1), length, bias16)
    return y.reshape(B, 1)


# speedup vs baseline: 2.0620x; 1.5710x over previous
import functools

import jax
import jax.numpy as jnp
from jax import lax
from jax.experimental import pallas as pl
from jax.experimental.pallas import tpu as pltpu
from jax.experimental.pallas import tpu_sc as plsc

B = 16384
L = 200
VOCAB = 7800
DIM = 64
VPAD = 7808
NC, NS = 2, 16
NW = NC * NS
RPW = B // NW
GROUPS = RPW // 16
TR = L // 8


def _project_body(e_ref, w_ref, o_ref):
    o_ref[...] = jnp.sum(e_ref[...] * w_ref[...], axis=1, keepdims=True)


def _project(table_pad, lin_w):
    return pl.pallas_call(
        _project_body,
        out_shape=jax.ShapeDtypeStruct((VPAD, 1), jnp.float32),
    )(table_pad, lin_w)


_MESH = plsc.VectorSubcoreMesh(core_axis_name="c", subcore_axis_name="s")


@functools.partial(
    pl.kernel,
    out_type=jax.ShapeDtypeStruct((B,), jnp.float32),
    mesh=_MESH,
    compiler_params=pltpu.CompilerParams(needs_layout_passes=False),
    scratch_types=[
        pltpu.VMEM((VPAD,), jnp.float32),
        pltpu.VMEM((32 * TR, 128), jnp.int32),
        pltpu.VMEM((RPW,), jnp.float32),
        pltpu.VMEM((16,), jnp.float32),
        pltpu.VMEM((RPW,), jnp.float32),
        pltpu.SemaphoreType.DMA,
    ],
)
def _sc_pool(s_hbm, x_hbm, len_hbm, bias_hbm, out_hbm,
             s_v, x_v, len_v, bias_v, out_v, sem):
    wid = lax.axis_index("s") * NC + lax.axis_index("c")
    base = wid * RPW
    pltpu.sync_copy(s_hbm, s_v)
    pltpu.sync_copy(len_hbm.at[pl.ds(base, RPW)], len_v)
    pltpu.sync_copy(bias_hbm, bias_v)

    copies = [
        pltpu.async_copy(
            x_hbm.at[pl.ds(tr * 1024 + 32 * wid, 32), :],
            x_v.at[pl.ds(tr * 32, 32), :], sem)
        for tr in range(TR)
    ]
    for cp in copies:
        cp.wait()

    lanes = lax.iota(jnp.int32, 16)
    bias = bias_v[...]
    zero = jnp.zeros((16,), jnp.float32)

    def group_body(g, carry):
        i0_init = jnp.zeros((16,), jnp.int32) + (g >> 3) * 8
        cvec = (g & 7) * 16 + lanes

        def band(tr, st):
            acc0, acc1, i0 = st
            for r in range(8):
                xv = plsc.load_gather(x_v, [i0 + r, cvec])
                val = plsc.load_gather(s_v, [xv])
                if r % 2 == 0:
                    acc0 = acc0 + val
                else:
                    acc1 = acc1 + val
            return acc0, acc1, i0 + 32

        a0, a1, _ = lax.fori_loop(0, TR, band, (zero, zero, i0_init))
        z = (a0 + a1) / len_v[pl.ds(g * 16, 16)] + bias
        out_v[pl.ds(g * 16, 16)] = 1.0 / (1.0 + jnp.exp(-z))
        return carry

    lax.fori_loop(0, GROUPS, group_body, 0)
    pltpu.sync_copy(out_v, out_hbm.at[pl.ds(base, RPW)])


def kernel(x, length, embed_table, lin_w, lin_b):
    table_pad = jnp.pad(embed_table, ((0, VPAD - VOCAB), (0, 0)))
    s = _project(table_pad, lin_w).reshape(VPAD)
    bias16 = jnp.broadcast_to(lin_b, (16,)).astype(jnp.float32)
    xt = (x.T.reshape(TR, 8, B // 128, 128)
          .transpose(0, 2, 1, 3)
          .reshape(TR * (B // 128) * 8, 128))
    y = _sc_pool(s, xt, length, bias16)
    return y.reshape(B, 1)


# speedup vs baseline: 2.2043x; 1.0690x over previous
import functools

import jax
import jax.numpy as jnp
from jax import lax
from jax.experimental import pallas as pl
from jax.experimental.pallas import tpu as pltpu
from jax.experimental.pallas import tpu_sc as plsc

B = 16384
L = 200
VOCAB = 7800
DIM = 64
VPAD = 7808
NC, NS = 2, 16
NW = NC * NS
RPW = B // NW
GROUPS = RPW // 16
TR = L // 8


def _project_body(e_ref, w_ref, o_ref):
    o_ref[...] = jnp.sum(e_ref[...] * w_ref[...], axis=1, keepdims=True)


def _project(table_pad, lin_w):
    return pl.pallas_call(
        _project_body,
        out_shape=jax.ShapeDtypeStruct((VPAD, 1), jnp.float32),
    )(table_pad, lin_w)


_MESH = plsc.VectorSubcoreMesh(core_axis_name="c", subcore_axis_name="s")


@functools.partial(
    pl.kernel,
    out_type=jax.ShapeDtypeStruct((B,), jnp.float32),
    mesh=_MESH,
    compiler_params=pltpu.CompilerParams(needs_layout_passes=False),
    scratch_types=[
        pltpu.VMEM((VPAD,), jnp.float32),
        pltpu.VMEM((8 * TR, 128), jnp.int32),
        pltpu.VMEM((8 * TR, 128), jnp.int32),
        pltpu.VMEM((RPW,), jnp.float32),
        pltpu.VMEM((16,), jnp.float32),
        pltpu.VMEM((RPW,), jnp.float32),
        pltpu.SemaphoreType.DMA,
        pltpu.SemaphoreType.DMA,
    ],
)
def _sc_pool(s_hbm, x_hbm, len_hbm, bias_hbm, out_hbm,
             s_v, xb0, xb1, len_v, bias_v, out_v, sem0, sem1):
    wid = lax.axis_index("s") * NC + lax.axis_index("c")
    base = wid * RPW
    bufs = (xb0, xb1)
    sems = (sem0, sem1)

    def start(tc):
        buf, sem = bufs[tc % 2], sems[tc % 2]
        return [
            pltpu.async_copy(
                x_hbm.at[pl.ds(tr * 1024 + 32 * wid + 8 * tc, 8), :],
                buf.at[pl.ds(tr * 8, 8), :], sem)
            for tr in range(TR)
        ]

    cps = start(0)
    pltpu.sync_copy(s_hbm, s_v)
    pltpu.sync_copy(len_hbm.at[pl.ds(base, RPW)], len_v)
    pltpu.sync_copy(bias_hbm, bias_v)

    lanes = lax.iota(jnp.int32, 16)
    bias = bias_v[...]
    zero = jnp.zeros((16,), jnp.float32)
    zero_i = jnp.zeros((16,), jnp.int32)

    for tc in range(4):
        for cp in cps:
            cp.wait()
        if tc + 1 < 4:
            cps = start(tc + 1)
        x_v = bufs[tc % 2]

        def group_body(j, carry):
            cvec = j * 16 + lanes

            def band(tr, st):
                acc0, acc1, i0 = st
                for r in range(8):
                    xv = plsc.load_gather(x_v, [i0 + r, cvec])
                    val = plsc.load_gather(s_v, [xv])
                    if r % 2 == 0:
                        acc0 = acc0 + val
                    else:
                        acc1 = acc1 + val
                return acc0, acc1, i0 + 8

            a0, a1, _ = lax.fori_loop(0, TR, band, (zero, zero, zero_i))
            off = tc * 128 + j * 16
            z = (a0 + a1) / len_v[pl.ds(off, 16)] + bias
            out_v[pl.ds(off, 16)] = 1.0 / (1.0 + jnp.exp(-z))
            return carry

        lax.fori_loop(0, 8, group_body, 0)
    pltpu.sync_copy(out_v, out_hbm.at[pl.ds(base, RPW)])


def kernel(x, length, embed_table, lin_w, lin_b):
    table_pad = jnp.pad(embed_table, ((0, VPAD - VOCAB), (0, 0)))
    s = _project(table_pad, lin_w).reshape(VPAD)
    bias16 = jnp.broadcast_to(lin_b, (16,)).astype(jnp.float32)
    xt = (x.T.reshape(TR, 8, B // 128, 128)
          .transpose(0, 2, 1, 3)
          .reshape(TR * (B // 128) * 8, 128))
    y = _sc_pool(s, xt, length, bias16)
    return y.reshape(B, 1)
